# Initial kernel scaffold; baseline (speedup 1.0000x reference)
import jax
import jax.numpy as jnp
from jax.experimental import pallas as pl


def kernel(x, Wr, W1, b1, W2, b2):
    raise NotImplementedError("write your pallas kernel here")


# speedup vs baseline: 1.1077x; 1.1077x over previous
import functools
import jax
import jax.numpy as jnp
from jax.experimental import pallas as pl
from jax.experimental.pallas import tpu as pltpu

T = 2048
D_MODEL = 768
D_FF = 3072
E = 8
TOPK = 2

_INTERPRET = False


def _router_body(x_ref, wr_ref, comb_ref):
    x = x_ref[...]
    wr = wr_ref[...]
    logits = jax.lax.dot_general(x, wr, (((1,), (0,)), ((), ())),
                                 preferred_element_type=jnp.float32)
    m = jnp.max(logits, axis=-1, keepdims=True)
    p = jnp.exp(logits - m)
    probs = p / jnp.sum(p, axis=-1, keepdims=True)
    e_iota = jax.lax.broadcasted_iota(jnp.int32, probs.shape, 1)
    v1 = jnp.max(probs, axis=-1, keepdims=True)
    i1 = jnp.min(jnp.where(probs == v1, e_iota, E), axis=-1, keepdims=True)
    masked = jnp.where(e_iota == i1, -1.0, probs)
    v2 = jnp.max(masked, axis=-1, keepdims=True)
    i2 = jnp.min(jnp.where(masked == v2, e_iota, E), axis=-1, keepdims=True)
    denom = v1 + v2 + 1e-9
    comb_ref[...] = (jnp.where(e_iota == i1, v1 / denom, 0.0)
                     + jnp.where(e_iota == i2, v2 / denom, 0.0))


def _ffn_body(comb_ref, x_ref, w1_ref, b1_ref, w2_ref, b2_ref, out_ref, acc_ref):
    e = pl.program_id(0)
    t = pl.program_id(1)
    TB = x_ref.shape[0]
    x = x_ref[...].astype(jnp.bfloat16)
    w1 = w1_ref[0].astype(jnp.bfloat16)
    w2 = w2_ref[0].astype(jnp.bfloat16)
    h = jax.lax.dot_general(x, w1, (((1,), (0,)), ((), ())),
                            preferred_element_type=jnp.float32) + b1_ref[0]
    h = jax.nn.gelu(h).astype(jnp.bfloat16)
    y = jax.lax.dot_general(h, w2, (((1,), (0,)), ((), ())),
                            preferred_element_type=jnp.float32) + b2_ref[0]
    comb = comb_ref[...]
    e_iota = jax.lax.broadcasted_iota(jnp.int32, comb.shape, 1)
    g = jnp.sum(jnp.where(e_iota == e, comb, 0.0), axis=-1, keepdims=True)
    contrib = g * y

    acc_slice = (pl.ds(t * TB, TB), slice(None))
    val = jnp.where(e == 0, contrib, acc_ref[acc_slice] + contrib)
    acc_ref[acc_slice] = val

    @pl.when(e == E - 1)
    def _():
        out_ref[...] = val


def kernel(x, Wr, W1, b1, W2, b2):
    combine = pl.pallas_call(
        _router_body,
        out_shape=jax.ShapeDtypeStruct((T, E), jnp.float32),
        interpret=_INTERPRET,
    )(x, Wr)

    TB = 256
    grid = (E, T // TB)
    out = pl.pallas_call(
        _ffn_body,
        grid=grid,
        in_specs=[
            pl.BlockSpec((TB, E), lambda e, t: (t, 0)),
            pl.BlockSpec((TB, D_MODEL), lambda e, t: (t, 0)),
            pl.BlockSpec((1, D_MODEL, D_FF), lambda e, t: (e, 0, 0)),
            pl.BlockSpec((1, 1, D_FF), lambda e, t: (e, 0, 0)),
            pl.BlockSpec((1, D_FF, D_MODEL), lambda e, t: (e, 0, 0)),
            pl.BlockSpec((1, 1, D_MODEL), lambda e, t: (e, 0, 0)),
        ],
        out_specs=pl.BlockSpec((TB, D_MODEL), lambda e, t: (t, 0)),
        out_shape=jax.ShapeDtypeStruct((T, D_MODEL), jnp.float32),
        scratch_shapes=[pltpu.VMEM((T, D_MODEL), jnp.float32)],
        compiler_params=pltpu.CompilerParams(
            dimension_semantics=("arbitrary", "arbitrary"),
        ),
        interpret=_INTERPRET,
    )(combine, x, W1, b1.reshape(E, 1, D_FF), W2, b2.reshape(E, 1, D_MODEL))
    return out


# speedup vs baseline: 1.4087x; 1.2717x over previous
import functools
import jax
import jax.numpy as jnp
from jax import lax
from jax.experimental import pallas as pl
from jax.experimental.pallas import tpu as pltpu
from jax.experimental.pallas import tpu_sc as plsc

T = 2048
D_MODEL = 768
D_FF = 3072
E = 8
TOPK = 2

B = 256
NTOT = T * TOPK + E * B
NB = NTOT // B

NC, NS = 2, 16
NW = NC * NS

_INTERPRET = False


def _router_body(x_ref, wr_ref, idx_ref, val_ref):
    x = x_ref[...]
    wr = wr_ref[...]
    logits = lax.dot_general(x, wr, (((1,), (0,)), ((), ())),
                             preferred_element_type=jnp.float32)
    m = jnp.max(logits, axis=-1, keepdims=True)
    p = jnp.exp(logits - m)
    probs = p / jnp.sum(p, axis=-1, keepdims=True)
    e_iota = lax.broadcasted_iota(jnp.int32, probs.shape, 1)
    v1 = jnp.max(probs, axis=-1, keepdims=True)
    i1 = jnp.min(jnp.where(probs == v1, e_iota, E), axis=-1, keepdims=True)
    masked = jnp.where(e_iota == i1, -1.0, probs)
    v2 = jnp.max(masked, axis=-1, keepdims=True)
    i2 = jnp.min(jnp.where(masked == v2, e_iota, E), axis=-1, keepdims=True)
    denom = v1 + v2 + 1e-9
    idx_ref[...] = jnp.concatenate([i1, i2], axis=1)
    val_ref[...] = jnp.concatenate([v1 / denom, v2 / denom], axis=1)


def _dispatch_body(x_hbm, q0_hbm, q1_hbm, g0_hbm, g1_hbm, xg_hbm, gs_hbm,
                   rows_v, q0_v, q1_v, g0_v, g1_v, sem):
    wid = lax.axis_index("s") * NC + lax.axis_index("c")
    base = wid * (T // NW)
    pltpu.sync_copy(x_hbm.at[pl.ds(base, 64)], rows_v)
    pltpu.sync_copy(q0_hbm.at[pl.ds(base, 64)], q0_v)
    pltpu.sync_copy(q1_hbm.at[pl.ds(base, 64)], q1_v)
    pltpu.sync_copy(g0_hbm.at[pl.ds(base, 64)], g0_v)
    pltpu.sync_copy(g1_hbm.at[pl.ds(base, 64)], g1_v)
    cp0 = pltpu.async_copy(rows_v, xg_hbm.at[q0_v], sem)
    cp1 = pltpu.async_copy(rows_v, xg_hbm.at[q1_v], sem)
    cp2 = pltpu.async_copy(g0_v, gs_hbm.at[q0_v], sem)
    cp3 = pltpu.async_copy(g1_v, gs_hbm.at[q1_v], sem)
    cp0.wait()
    cp1.wait()
    cp2.wait()
    cp3.wait()


def _ffn_body(meta_ref, xg_ref, gs_ref, w1_ref, b1_ref, w2_ref, b2_ref,
              out_ref):
    b = pl.program_id(0)

    @pl.when(b < meta_ref[NB])
    def _():
        x = xg_ref[...].astype(jnp.bfloat16)
        w1 = w1_ref[0].astype(jnp.bfloat16)
        w2 = w2_ref[0].astype(jnp.bfloat16)
        h = lax.dot_general(x, w1, (((1,), (0,)), ((), ())),
                            preferred_element_type=jnp.float32) + b1_ref[0]
        h = jax.nn.gelu(h).astype(jnp.bfloat16)
        y = lax.dot_general(h, w2, (((1,), (0,)), ((), ())),
                            preferred_element_type=jnp.float32) + b2_ref[0]
        out_ref[...] = y * gs_ref[...]


def _combine_body(y_hbm, q0_hbm, q1_hbm, out_hbm,
                  r0_v, r1_v, q0_v, q1_v, sem0, sem1):
    wid = lax.axis_index("s") * NC + lax.axis_index("c")
    tpw = T // NW
    CH = 32

    def chunk(i, _):
        base = wid * tpw + i * CH
        pltpu.sync_copy(q0_hbm.at[pl.ds(base, CH)], q0_v)
        pltpu.sync_copy(q1_hbm.at[pl.ds(base, CH)], q1_v)
        cp0 = pltpu.async_copy(y_hbm.at[q0_v], r0_v, sem0)
        cp1 = pltpu.async_copy(y_hbm.at[q1_v], r1_v, sem1)
        cp0.wait()
        cp1.wait()

        def row(j, _):
            for c in range(D_MODEL // 16):
                sl = pl.ds(c * 16, 16)
                r0_v[j, sl] = r0_v[j, sl] + r1_v[j, sl]
            return 0

        lax.fori_loop(0, CH, row, 0)
        pltpu.sync_copy(r0_v, out_hbm.at[pl.ds(base, CH)])
        return 0

    lax.fori_loop(0, tpw // CH, chunk, 0)


def kernel(x, Wr, W1, b1, W2, b2):
    gate_idx, gate_vals = pl.pallas_call(
        _router_body,
        out_shape=(jax.ShapeDtypeStruct((T, TOPK), jnp.int32),
                   jax.ShapeDtypeStruct((T, TOPK), jnp.float32)),
        interpret=_INTERPRET,
    )(x, Wr)

    ef = gate_idx.reshape(-1)
    oh = (ef[:, None] == jnp.arange(E)[None, :]).astype(jnp.int32)
    cnt = jnp.sum(oh, axis=0)
    rank = jnp.take_along_axis(jnp.cumsum(oh, axis=0) - oh,
                               ef[:, None], axis=1)[:, 0]
    pc = ((cnt + B - 1) // B) * B
    cum = jnp.cumsum(pc)
    poff = cum - pc
    q = poff[ef] + rank
    q0 = q[0::2].astype(jnp.int32)
    q1 = q[1::2].astype(jnp.int32)
    block_expert = jnp.minimum(
        jnp.searchsorted(cum, jnp.arange(NB, dtype=jnp.int32) * B,
                         side="right").astype(jnp.int32), E - 1)
    nactive = (cum[-1] // B).astype(jnp.int32)
    meta = jnp.concatenate([block_expert, nactive[None]])
    g0 = gate_vals[:, 0]
    g1 = gate_vals[:, 1]

    mesh = plsc.VectorSubcoreMesh(core_axis_name="c", subcore_axis_name="s")

    xg, gs = pl.kernel(
        _dispatch_body,
        out_type=(jax.ShapeDtypeStruct((NTOT, D_MODEL), jnp.float32),
                  jax.ShapeDtypeStruct((NTOT,), jnp.float32)),
        mesh=mesh,
        scratch_types=[
            pltpu.VMEM((64, D_MODEL), jnp.float32),
            pltpu.VMEM((64,), jnp.int32),
            pltpu.VMEM((64,), jnp.int32),
            pltpu.VMEM((64,), jnp.float32),
            pltpu.VMEM((64,), jnp.float32),
            pltpu.SemaphoreType.DMA,
        ],
    )(x, q0, q1, g0, g1)

    y = pl.pallas_call(
        _ffn_body,
        grid_spec=pltpu.PrefetchScalarGridSpec(
            num_scalar_prefetch=1,
            grid=(NB,),
            in_specs=[
                pl.BlockSpec((B, D_MODEL), lambda b, m: (b, 0)),
                pl.BlockSpec((B, 1), lambda b, m: (b, 0)),
                pl.BlockSpec((1, D_MODEL, D_FF), lambda b, m: (m[b], 0, 0)),
                pl.BlockSpec((1, 1, D_FF), lambda b, m: (m[b], 0, 0)),
                pl.BlockSpec((1, D_FF, D_MODEL), lambda b, m: (m[b], 0, 0)),
                pl.BlockSpec((1, 1, D_MODEL), lambda b, m: (m[b], 0, 0)),
            ],
            out_specs=pl.BlockSpec((B, D_MODEL), lambda b, m: (b, 0)),
        ),
        out_shape=jax.ShapeDtypeStruct((NTOT, D_MODEL), jnp.float32),
        compiler_params=pltpu.CompilerParams(
            dimension_semantics=("arbitrary",),
        ),
        interpret=_INTERPRET,
    )(meta, xg, gs.reshape(NTOT, 1), W1, b1.reshape(E, 1, D_FF), W2,
      b2.reshape(E, 1, D_MODEL))

    out = pl.kernel(
        _combine_body,
        out_type=jax.ShapeDtypeStruct((T, D_MODEL), jnp.float32),
        mesh=mesh,
        scratch_types=[
            pltpu.VMEM((32, D_MODEL), jnp.float32),
            pltpu.VMEM((32, D_MODEL), jnp.float32),
            pltpu.VMEM((32,), jnp.int32),
            pltpu.VMEM((32,), jnp.int32),
            pltpu.SemaphoreType.DMA,
            pltpu.SemaphoreType.DMA,
        ],
    )(y, q0, q1)

    return out
